# baseline (speedup 1.0000x reference)
import functools

import numpy as np

import jax
import jax.numpy as jnp
from jax import lax
from jax.experimental import pallas as pl
from jax.experimental.pallas import tpu as pltpu
from jax.experimental.pallas import tpu_sc as plsc

BATCH = 16384
HIST = 50
EMBED_DIM = 64

NC = 2
NS = 16
NW = NC * NS

LANES = 128
J_BLOCKS = BATCH // LANES
J_PER_W = J_BLOCKS // NW
B_PER_W = BATCH // NW
T_PER_W = HIST * J_PER_W
N_TILES = HIST * 8 * LANES
NBUF = 4


def _make_kernel():
  mesh = plsc.VectorSubcoreMesh(core_axis_name="c", subcore_axis_name="s")

  @functools.partial(
      pl.kernel,
      mesh=mesh,
      out_type=jax.ShapeDtypeStruct((N_TILES * 8 * LANES,), jnp.float32),
      compiler_params=pltpu.CompilerParams(
          use_tc_tiling_on_sc=False, needs_layout_passes=False),
      scratch_types=[
          pltpu.VMEM((HIST, B_PER_W), jnp.int32),
          [pltpu.VMEM((LANES, EMBED_DIM), jnp.float32) for _ in range(NBUF)],
          [pltpu.VMEM((8 * 8 * LANES,), jnp.float32) for _ in range(NBUF)],
          [pltpu.SemaphoreType.DMA for _ in range(NBUF)],
          [pltpu.SemaphoreType.DMA for _ in range(NBUF)],
      ],
  )
  def body(table_hbm, idxt_hbm, out_hbm, idx_v, rows, t3, gsem, osem):
    wid = lax.axis_index("s") * NC + lax.axis_index("c")
    jbase = wid * J_PER_W

    pltpu.sync_copy(idxt_hbm.at[:, pl.ds(wid * B_PER_W, B_PER_W)], idx_v)

    def task_hj(t):
      return t // J_PER_W, t % J_PER_W

    def gather(t, b):
      h, j = task_hj(t)
      pltpu.async_copy(
          table_hbm.at[idx_v.at[h, pl.ds(j * LANES, LANES)]], rows[b], gsem[b])

    def wait_gather(t, b):
      h, j = task_hj(t)
      pltpu.make_async_copy(
          table_hbm.at[idx_v.at[h, pl.ds(j * LANES, LANES)]], rows[b],
          gsem[b]).wait()

    def transpose(b):
      iota = lax.iota(jnp.int32, 16)
      base = [(iota + c0) * LANES for c0 in range(0, 64, 16)]

      @plsc.parallel_loop(0, LANES, 1, unroll=8)
      def col(l):
        splat_l = jnp.full((16,), 0, jnp.int32) + l
        for ci, c0 in enumerate(range(0, 64, 16)):
          v = rows[b][l, pl.ds(c0, 16)]
          plsc.store_scatter(t3[b], [base[ci] + splat_l], v)

    def put(t, b):
      h, j = task_hj(t)
      for i in range(8):
        pltpu.async_copy(
            t3[b].at[pl.ds(i * 8 * LANES, 8 * LANES)],
            out_hbm.at[pl.ds(((h * 8 + i) * LANES + jbase + j) * 8 * LANES,
                             8 * LANES)],
            osem[b])

    def wait_put(t, b):
      h, j = task_hj(t)
      for i in range(8):
        pltpu.make_async_copy(
            t3[b].at[pl.ds(i * 8 * LANES, 8 * LANES)],
            out_hbm.at[pl.ds(((h * 8 + i) * LANES + jbase + j) * 8 * LANES,
                             8 * LANES)],
            osem[b]).wait()

    for b in range(NBUF):
      gather(b, b)

    for b in range(NBUF):
      wait_gather(b, b)
      transpose(b)
      put(b, b)
      gather(b + NBUF, b)

    n_rounds = T_PER_W // NBUF

    def step(r, carry):
      for b in range(NBUF):
        t = r * NBUF + b
        wait_gather(t, b)
        wait_put(t - NBUF, b)
        transpose(b)
        put(t, b)
        gather(t + NBUF, b)
      return carry

    lax.fori_loop(1, n_rounds - 1, step, 0)

    for b in range(NBUF):
      t = T_PER_W - NBUF + b
      wait_gather(t, b)
      wait_put(t - NBUF, b)
      transpose(b)
      put(t, b)

    for b in range(NBUF):
      wait_put(T_PER_W - NBUF + b, b)

  return body


_kernel = _make_kernel()


def kernel(input, table):
  idxt = input.T.astype(jnp.int32)
  out = _kernel(table, idxt)
  a = out.reshape(HIST, 8, LANES, 8, LANES)
  a = a.transpose(2, 4, 0, 1, 3)
  return a.reshape(BATCH, HIST, EMBED_DIM)


# speedup vs baseline: 1.4260x; 1.4260x over previous
import functools

import numpy as np

import jax
import jax.numpy as jnp
from jax import lax
from jax.experimental import pallas as pl
from jax.experimental.pallas import tpu as pltpu
from jax.experimental.pallas import tpu_sc as plsc

BATCH = 16384
HIST = 50
EMBED_DIM = 64

NC = 2
NS = 16
NW = NC * NS

LANES = 128
J_BLOCKS = BATCH // LANES
J_PER_W = J_BLOCKS // NW
B_PER_W = BATCH // NW
T_PER_W = HIST * J_PER_W
N_TILES = HIST * 8 * LANES
NBUF = 2


def _make_kernel():
  mesh = plsc.VectorSubcoreMesh(core_axis_name="c", subcore_axis_name="s")

  @functools.partial(
      pl.kernel,
      mesh=mesh,
      out_type=jax.ShapeDtypeStruct((N_TILES * 8 * LANES,), jnp.float32),
      compiler_params=pltpu.CompilerParams(
          use_tc_tiling_on_sc=False, needs_layout_passes=False),
      scratch_types=[
          pltpu.VMEM((HIST, B_PER_W), jnp.int32),
          [pltpu.VMEM((LANES, EMBED_DIM), jnp.float32) for _ in range(NBUF)],
          [pltpu.VMEM((8 * 8 * LANES,), jnp.float32) for _ in range(NBUF)],
          [pltpu.SemaphoreType.DMA for _ in range(NBUF)],
          [pltpu.SemaphoreType.DMA for _ in range(NBUF)],
      ],
  )
  def body(table_hbm, idxt_hbm, out_hbm, idx_v, rows, t3, gsem, osem):
    wid = lax.axis_index("s") * NC + lax.axis_index("c")
    jbase = wid * J_PER_W

    pltpu.sync_copy(idxt_hbm.at[:, pl.ds(wid * B_PER_W, B_PER_W)], idx_v)

    def task_hj(t):
      return t // J_PER_W, t % J_PER_W

    def gather(t, b):
      h, j = task_hj(t)
      pltpu.async_copy(
          table_hbm.at[idx_v.at[h, pl.ds(j * LANES, LANES)]], rows[b], gsem[b])

    def wait_gather(t, b):
      h, j = task_hj(t)
      pltpu.make_async_copy(
          table_hbm.at[idx_v.at[h, pl.ds(j * LANES, LANES)]], rows[b],
          gsem[b]).wait()

    iota = lax.iota(jnp.int32, 16)
    rot = [lax.bitwise_and(iota + d, 15) for d in range(16)]
    dstb = [rot[d] * LANES + iota for d in range(16)]

    def transpose(b):
      @plsc.parallel_loop(0, LANES, 16, unroll=1)
      def blk(l0):
        idx_row = iota + l0
        for c0 in range(0, 64, 16):
          for d in range(16):
            v = plsc.load_gather(rows[b], [idx_row, rot[d] + c0])
            plsc.store_scatter(t3[b], [dstb[d] + (c0 * LANES + l0)], v)

    def put(t, b):
      h, j = task_hj(t)
      for i in range(8):
        pltpu.async_copy(
            t3[b].at[pl.ds(i * 8 * LANES, 8 * LANES)],
            out_hbm.at[pl.ds(((h * 8 + i) * LANES + jbase + j) * 8 * LANES,
                             8 * LANES)],
            osem[b])

    def wait_put(t, b):
      h, j = task_hj(t)
      for i in range(8):
        pltpu.make_async_copy(
            t3[b].at[pl.ds(i * 8 * LANES, 8 * LANES)],
            out_hbm.at[pl.ds(((h * 8 + i) * LANES + jbase + j) * 8 * LANES,
                             8 * LANES)],
            osem[b]).wait()

    for b in range(NBUF):
      gather(b, b)

    for b in range(NBUF):
      wait_gather(b, b)
      transpose(b)
      put(b, b)
      gather(b + NBUF, b)

    n_rounds = T_PER_W // NBUF

    def step(r, carry):
      for b in range(NBUF):
        t = r * NBUF + b
        wait_gather(t, b)
        wait_put(t - NBUF, b)
        transpose(b)
        put(t, b)
        gather(t + NBUF, b)
      return carry

    lax.fori_loop(1, n_rounds - 1, step, 0)

    for b in range(NBUF):
      t = T_PER_W - NBUF + b
      wait_gather(t, b)
      wait_put(t - NBUF, b)
      transpose(b)
      put(t, b)

    for b in range(NBUF):
      wait_put(T_PER_W - NBUF + b, b)

  return body


_kernel = _make_kernel()


def kernel(input, table):
  idxt = input.T.astype(jnp.int32)
  out = _kernel(table, idxt)
  a = out.reshape(HIST, 8, LANES, 8, LANES)
  a = a.transpose(2, 4, 0, 1, 3)
  return a.reshape(BATCH, HIST, EMBED_DIM)


# speedup vs baseline: 1.4764x; 1.0354x over previous
import functools

import numpy as np

import jax
import jax.numpy as jnp
from jax import lax
from jax.experimental import pallas as pl
from jax.experimental.pallas import tpu as pltpu
from jax.experimental.pallas import tpu_sc as plsc

BATCH = 16384
HIST = 50
EMBED_DIM = 64

NC = 2
NS = 16
NW = NC * NS

LANES = 128
J_BLOCKS = BATCH // LANES
J_PER_W = J_BLOCKS // NW
B_PER_W = BATCH // NW
T_PER_W = HIST * J_PER_W
N_TILES = HIST * 8 * LANES
NBUF = 2


def _make_kernel():
  mesh = plsc.VectorSubcoreMesh(core_axis_name="c", subcore_axis_name="s")

  @functools.partial(
      pl.kernel,
      mesh=mesh,
      out_type=jax.ShapeDtypeStruct((N_TILES * 8 * LANES,), jnp.float32),
      compiler_params=pltpu.CompilerParams(
          use_tc_tiling_on_sc=False, needs_layout_passes=False),
      scratch_types=[
          pltpu.VMEM((HIST, B_PER_W), jnp.int32),
          [pltpu.VMEM((LANES, EMBED_DIM), jnp.float32) for _ in range(NBUF)],
          [pltpu.VMEM((8 * 8 * LANES,), jnp.float32) for _ in range(NBUF)],
          [pltpu.SemaphoreType.DMA for _ in range(NBUF)],
          [pltpu.SemaphoreType.DMA for _ in range(NBUF)],
      ],
  )
  def body(table_hbm, idxt_hbm, out_hbm, idx_v, rows, t3, gsem, osem):
    wid = lax.axis_index("s") * NC + lax.axis_index("c")
    jbase = wid * J_PER_W

    pltpu.sync_copy(idxt_hbm.at[:, pl.ds(wid * B_PER_W, B_PER_W)], idx_v)

    def task_hj(t):
      return t // J_PER_W, t % J_PER_W

    def gather(t, b):
      h, j = task_hj(t)
      pltpu.async_copy(
          table_hbm.at[idx_v.at[h, pl.ds(j * LANES, LANES)]], rows[b], gsem[b])

    def wait_gather(t, b):
      h, j = task_hj(t)
      pltpu.make_async_copy(
          table_hbm.at[idx_v.at[h, pl.ds(j * LANES, LANES)]], rows[b],
          gsem[b]).wait()

    iota = lax.iota(jnp.int32, 16)
    rot = [lax.bitwise_and(iota + d, 15) for d in range(16)]
    dstb = [rot[d] * LANES + iota for d in range(16)]

    def transpose(b):
      @plsc.parallel_loop(0, LANES, 16, unroll=2)
      def blk(l0):
        idx_row = iota + l0
        for c0 in range(0, 64, 16):
          for d in range(16):
            v = plsc.load_gather(rows[b], [idx_row, rot[d] + c0])
            plsc.store_scatter(t3[b], [dstb[d] + (c0 * LANES + l0)], v)

    def put(t, b):
      h, j = task_hj(t)
      for i in range(8):
        pltpu.async_copy(
            t3[b].at[pl.ds(i * 8 * LANES, 8 * LANES)],
            out_hbm.at[pl.ds(((h * 8 + i) * LANES + jbase + j) * 8 * LANES,
                             8 * LANES)],
            osem[b])

    def wait_put(t, b):
      h, j = task_hj(t)
      for i in range(8):
        pltpu.make_async_copy(
            t3[b].at[pl.ds(i * 8 * LANES, 8 * LANES)],
            out_hbm.at[pl.ds(((h * 8 + i) * LANES + jbase + j) * 8 * LANES,
                             8 * LANES)],
            osem[b]).wait()

    for b in range(NBUF):
      gather(b, b)

    for b in range(NBUF):
      wait_gather(b, b)
      transpose(b)
      put(b, b)
      gather(b + NBUF, b)

    n_rounds = T_PER_W // NBUF

    def step(r, carry):
      for b in range(NBUF):
        t = r * NBUF + b
        wait_gather(t, b)
        wait_put(t - NBUF, b)
        transpose(b)
        put(t, b)
        gather(t + NBUF, b)
      return carry

    lax.fori_loop(1, n_rounds - 1, step, 0)

    for b in range(NBUF):
      t = T_PER_W - NBUF + b
      wait_gather(t, b)
      wait_put(t - NBUF, b)
      transpose(b)
      put(t, b)

    for b in range(NBUF):
      wait_put(T_PER_W - NBUF + b, b)

  return body


_kernel = _make_kernel()


def kernel(input, table):
  idxt = input.T.astype(jnp.int32)
  out = _kernel(table, idxt)
  a = out.reshape(HIST, 8, LANES, 8, LANES)
  a = a.transpose(2, 4, 0, 1, 3)
  return a.reshape(BATCH, HIST, EMBED_DIM)
